# baseline (speedup 1.0000x reference)
import functools

import jax
import jax.numpy as jnp
from jax import lax
from jax.experimental import pallas as pl
from jax.experimental.pallas import tpu as pltpu
from jax.experimental.pallas import tpu_sc as plsc

N = 16384
D = 2304
R = 53
NB = 1024

BN = 2048
G = N // BN
TW = 64

NW = 32
RW = N // NW
CH = 128
NCH = RW // CH

_PREC = jax.lax.Precision.DEFAULT


def _tc1_kernel(x_ref, rel_ref, query_ref, starts_ref, out_tbl_ref,
                out_gmax_ref, gmax_ref):
    i = pl.program_id(0)

    s_blk = jax.lax.dot_general(
        x_ref[...], rel_ref[...],
        dimension_numbers=(((1,), (0,)), ((), ())),
        preferred_element_type=jnp.float32, precision=_PREC)
    q = query_ref[...]
    rel_ids = jax.lax.broadcasted_iota(jnp.int32, (BN, R), 1)
    att = jnp.sum(jnp.where(rel_ids == q, s_blk, 0.0), axis=1,
                  keepdims=True)
    idx = jax.lax.broadcasted_iota(jnp.int32, (BN, NB), 0) + i * BN
    seg = jnp.sum(jnp.where(idx >= starts_ref[...], 1.0, 0.0),
                  axis=1, keepdims=True) - 1.0
    out_tbl_ref[...] = jnp.concatenate(
        [s_blk, jnp.ones((BN, 1), jnp.float32), att, seg,
         jnp.zeros((BN, TW - R - 3), jnp.float32)], axis=1)

    blk_max = jnp.max(att)

    @pl.when(i == 0)
    def _():
        gmax_ref[0, 0] = blk_max

    @pl.when(i > 0)
    def _():
        gmax_ref[0, 0] = jnp.maximum(gmax_ref[0, 0], blk_max)

    @pl.when(i == G - 1)
    def _():
        out_gmax_ref[...] = jnp.full((1, 16), gmax_ref[0, 0], jnp.float32)


def _tc1(x, rel_mat, query_col, starts):
    return pl.pallas_call(
        _tc1_kernel,
        grid=(G,),
        in_specs=[
            pl.BlockSpec((BN, D), lambda i: (i, 0)),
            pl.BlockSpec((D, R), lambda i: (0, 0)),
            pl.BlockSpec((BN, 1), lambda i: (i, 0)),
            pl.BlockSpec((1, NB), lambda i: (0, 0)),
        ],
        out_specs=[
            pl.BlockSpec((BN, TW), lambda i: (i, 0)),
            pl.BlockSpec((1, 16), lambda i: (0, 0)),
        ],
        out_shape=[
            jax.ShapeDtypeStruct((N, TW), jnp.float32),
            jax.ShapeDtypeStruct((1, 16), jnp.float32),
        ],
        scratch_shapes=[pltpu.SMEM((1, 1), jnp.float32)],
    )(x, rel_mat, query_col, starts)


def _sc_body(tbl_hbm, gmax_hbm, out_hbm, buf0_v, buf1_v, ew_v, gm_v, acc_v,
             sem0, sem1):
    wid = lax.axis_index("s") * 2 + lax.axis_index("c")
    z16 = jnp.zeros((16,), jnp.float32)
    lane = lax.iota(jnp.int32, 16)
    bufs = (buf0_v, buf1_v)
    sems = (sem0, sem1)

    def zero_body(j, _):
        base = j * 256
        for t in range(16):
            acc_v[pl.ds(base + t * 16, 16)] = z16
        return 0

    lax.fori_loop(0, NB * TW // 256, zero_body, 0)

    pltpu.sync_copy(gmax_hbm, gm_v)
    gm = gm_v[...]

    def chunk_src(k):
        return tbl_hbm.at[pl.ds((wid * RW + k * CH) * TW, CH * TW)]

    pltpu.async_copy(chunk_src(0), bufs[0], sems[0])
    for k in range(NCH):
        buf_v = bufs[k % 2]
        pltpu.make_async_copy(chunk_src(k), buf_v, sems[k % 2]).wait()
        if k + 1 < NCH:
            pltpu.async_copy(chunk_src(k + 1), bufs[(k + 1) % 2],
                             sems[(k + 1) % 2])

        def e_body(g, _):
            att = plsc.load_gather(
                buf_v, [lane * TW + (g * (16 * TW) + R + 1)])
            ew_v[pl.ds(g * 16, 16)] = jnp.exp(att - gm)
            return 0

        lax.fori_loop(0, CH // 16, e_body, 0)

        def row_body(r, _):
            e_spl = plsc.load_gather(ew_v, [jnp.full((16,), r, jnp.int32)])
            seg_f = plsc.load_gather(
                buf_v, [jnp.full((16,), r * TW + (R + 2), jnp.int32)])
            seg_i = seg_f.astype(jnp.int32)
            for c in range(TW // 16):
                val = buf_v[pl.ds(r * TW + c * 16, 16)]
                plsc.addupdate_scatter(
                    acc_v, [seg_i * TW + c * 16 + lane], val * e_spl)
            return 0

        lax.fori_loop(0, CH, row_body, 0)

    pltpu.sync_copy(acc_v, out_hbm.at[wid])


def _sc_stage(tbl_flat, gmax16):
    mesh = plsc.VectorSubcoreMesh(core_axis_name="c", subcore_axis_name="s")
    f = pl.kernel(
        _sc_body,
        mesh=mesh,
        compiler_params=pltpu.CompilerParams(needs_layout_passes=False),
        out_type=jax.ShapeDtypeStruct((NW, NB * TW), jnp.float32),
        scratch_types=[
            pltpu.VMEM((CH * TW,), jnp.float32),
            pltpu.VMEM((CH * TW,), jnp.float32),
            pltpu.VMEM((CH,), jnp.float32),
            pltpu.VMEM((16,), jnp.float32),
            pltpu.VMEM((NB * TW,), jnp.float32),
            pltpu.SemaphoreType.DMA,
            pltpu.SemaphoreType.DMA,
        ],
    )
    return f(tbl_flat, gmax16)


def _tc2_kernel(parts_ref, bias_ref, out_ref):
    acc = parts_ref[pl.ds(0, NB), :]
    for w in range(1, NW):
        acc = acc + parts_ref[pl.ds(w * NB, NB), :]
    out_ref[...] = acc[:, :R] / acc[:, R:R + 1] + bias_ref[...]


def _tc2(parts, bias_row):
    return pl.pallas_call(
        _tc2_kernel,
        in_specs=[
            pl.BlockSpec((NW * NB, TW), lambda: (0, 0)),
            pl.BlockSpec((1, R), lambda: (0, 0)),
        ],
        out_specs=pl.BlockSpec((NB, R), lambda: (0, 0)),
        out_shape=jax.ShapeDtypeStruct((NB, R), jnp.float32),
    )(parts, bias_row)


@jax.jit
def kernel(x, scope, query, rel_mat, bias):
    query_col = query.astype(jnp.int32).reshape(N, 1)
    starts = scope[:, 0].astype(jnp.int32).reshape(1, NB)
    bias_row = bias.reshape(1, R)

    tbl, gmax16 = _tc1(x, rel_mat, query_col, starts)
    parts = _sc_stage(tbl.reshape(N * TW), gmax16.reshape(16))
    return _tc2(parts.reshape(NW * NB, TW), bias_row)


# speedup vs baseline: 1.8317x; 1.8317x over previous
import functools

import jax
import jax.numpy as jnp
from jax.experimental import pallas as pl
from jax.experimental.pallas import tpu as pltpu

N = 16384
D = 2304
R = 53
NB = 1024

BN = 1024
G = N // BN
TW = 54

_PREC = jax.lax.Precision.DEFAULT


def _selector_kernel(x_ref, rel_ref, query_ref, starts_ref, ends_ref,
                     bias_ref, out_ref, acc_scr, gmax_ref):
    i = pl.program_id(0)

    s_blk = jax.lax.dot_general(
        x_ref[...], rel_ref[...],
        dimension_numbers=(((1,), (0,)), ((), ())),
        preferred_element_type=jnp.float32, precision=_PREC)
    q = query_ref[...]
    rel_ids = jax.lax.broadcasted_iota(jnp.int32, (BN, R), 1)
    att = jnp.sum(jnp.where(rel_ids == q, s_blk, 0.0), axis=1,
                  keepdims=True)
    tbl = jnp.concatenate(
        [s_blk, jnp.ones((BN, 1), jnp.float32)], axis=1)

    blk_max = jnp.max(att)
    idx = jax.lax.broadcasted_iota(jnp.int32, (BN, NB), 0) + i * BN
    mask = (idx >= starts_ref[...]) & (idx < ends_ref[...])

    def contrib(m):
        mbf = jnp.where(mask, jnp.float32(1.0), 0.0).astype(jnp.bfloat16)
        tbl_e = (tbl * jnp.exp(att - m)).astype(jnp.bfloat16)
        return jax.lax.dot_general(
            mbf, tbl_e, dimension_numbers=(((0,), (0,)), ((), ())),
            preferred_element_type=jnp.float32, precision=_PREC)

    @pl.when(i == 0)
    def _():
        gmax_ref[0, 0] = blk_max
        acc_scr[...] = contrib(blk_max)

    @pl.when(i > 0)
    def _():
        m_old = gmax_ref[0, 0]
        m_new = jnp.maximum(m_old, blk_max)
        gmax_ref[0, 0] = m_new
        acc_scr[...] = (acc_scr[...] * jnp.exp(m_old - m_new)
                        + contrib(m_new))

    @pl.when(i == G - 1)
    def _():
        acc = acc_scr[...]
        out_ref[...] = acc[:, :R] / acc[:, R:R + 1] + bias_ref[...]


@functools.partial(jax.jit, static_argnames=("interpret",))
def kernel(x, scope, query, rel_mat, bias, interpret=False):
    query_col = query.astype(jnp.int32).reshape(N, 1)
    starts = scope[:, 0].astype(jnp.int32).reshape(1, NB)
    ends = scope[:, 1].astype(jnp.int32).reshape(1, NB)
    bias_row = bias.reshape(1, R)

    return pl.pallas_call(
        _selector_kernel,
        grid=(G,),
        in_specs=[
            pl.BlockSpec((BN, D), lambda i: (i, 0)),
            pl.BlockSpec((D, R), lambda i: (0, 0)),
            pl.BlockSpec((BN, 1), lambda i: (i, 0)),
            pl.BlockSpec((1, NB), lambda i: (0, 0)),
            pl.BlockSpec((1, NB), lambda i: (0, 0)),
            pl.BlockSpec((1, R), lambda i: (0, 0)),
        ],
        out_specs=pl.BlockSpec((NB, R), lambda i: (0, 0)),
        out_shape=jax.ShapeDtypeStruct((NB, R), jnp.float32),
        scratch_shapes=[
            pltpu.VMEM((NB, TW), jnp.float32),
            pltpu.SMEM((1, 1), jnp.float32),
        ],
        interpret=interpret,
    )(x, rel_mat, query_col, starts, ends, bias_row)


# speedup vs baseline: 1.8337x; 1.0011x over previous
import jax
import jax.numpy as jnp
from jax.experimental import pallas as pl
from jax.experimental.pallas import tpu as pltpu

N = 16384
D = 2304
R = 53
NB = 1024

BN = 1024
G = N // BN
TW = 54

_PREC = jax.lax.Precision.DEFAULT


def _selector_kernel(x_ref, rel_ref, query_ref, starts_ref, ends_ref,
                     bias_ref, out_ref, acc_scr, gmax_ref):
    i = pl.program_id(0)

    s_blk = jax.lax.dot_general(
        x_ref[...], rel_ref[...],
        dimension_numbers=(((1,), (0,)), ((), ())),
        preferred_element_type=jnp.float32, precision=_PREC)
    q = query_ref[...]
    rel_ids = jax.lax.broadcasted_iota(jnp.int32, (BN, R), 1)
    att = jnp.sum(jnp.where(rel_ids == q, s_blk, 0.0), axis=1,
                  keepdims=True)
    tbl = jnp.concatenate(
        [s_blk, jnp.ones((BN, 1), jnp.float32)], axis=1)

    blk_max = jnp.max(att)
    idx = jax.lax.broadcasted_iota(jnp.int32, (BN, NB), 0) + i * BN
    mask = (idx >= starts_ref[...]) & (idx < ends_ref[...])

    def contrib(m):
        mbf = jnp.where(mask, jnp.float32(1.0), 0.0).astype(jnp.bfloat16)
        tbl_e = (tbl * jnp.exp(att - m)).astype(jnp.bfloat16)
        return jax.lax.dot_general(
            mbf, tbl_e, dimension_numbers=(((0,), (0,)), ((), ())),
            preferred_element_type=jnp.float32, precision=_PREC)

    @pl.when(i == 0)
    def _():
        gmax_ref[0, 0] = blk_max
        acc_scr[...] = contrib(blk_max)

    @pl.when(i > 0)
    def _():
        m_old = gmax_ref[0, 0]
        m_new = jnp.maximum(m_old, blk_max)
        gmax_ref[0, 0] = m_new
        acc_scr[...] = (acc_scr[...] * jnp.exp(m_old - m_new)
                        + contrib(m_new))

    @pl.when(i == G - 1)
    def _():
        acc = acc_scr[...]
        out_ref[...] = acc[:, :R] / acc[:, R:R + 1] + bias_ref[...]


@jax.jit
def kernel(x, scope, query, rel_mat, bias):
    query_col = query.astype(jnp.int32).reshape(N, 1)
    starts = scope[:, 0].astype(jnp.int32).reshape(1, NB)
    ends = scope[:, 1].astype(jnp.int32).reshape(1, NB)
    bias_row = bias.reshape(1, R)

    return pl.pallas_call(
        _selector_kernel,
        grid=(G,),
        in_specs=[
            pl.BlockSpec((BN, D), lambda i: (i, 0)),
            pl.BlockSpec((D, R), lambda i: (0, 0)),
            pl.BlockSpec((BN, 1), lambda i: (i, 0)),
            pl.BlockSpec((1, NB), lambda i: (0, 0)),
            pl.BlockSpec((1, NB), lambda i: (0, 0)),
            pl.BlockSpec((1, R), lambda i: (0, 0)),
        ],
        out_specs=pl.BlockSpec((NB, R), lambda i: (0, 0)),
        out_shape=jax.ShapeDtypeStruct((NB, R), jnp.float32),
        scratch_shapes=[
            pltpu.VMEM((NB, TW), jnp.float32),
            pltpu.SMEM((1, 1), jnp.float32),
        ],
    )(x, rel_mat, query_col, starts, ends, bias_row)
